# baseline (speedup 1.0000x reference)
import math

import jax
import jax.numpy as jnp
from jax.experimental import pallas as pl

_MAX_RATIO = abs(math.log(16.0 / 1000.0))

_BLOCK_N = 4000


def _decode_body(regress_ref, anchors_ref, out_ref):
    a = anchors_ref[...]
    d = regress_ref[0]
    a0 = a[:, 0:1]
    a1 = a[:, 1:2]
    a2 = a[:, 2:3]
    a3 = a[:, 3:4]
    dx = d[:, 0:1]
    dy = d[:, 1:2]
    dw = jnp.clip(d[:, 2:3], -_MAX_RATIO, _MAX_RATIO)
    dh = jnp.clip(d[:, 3:4], -_MAX_RATIO, _MAX_RATIO)
    w = a2 - a0
    h = a3 - a1
    cx = a0 + 0.5 * w
    cy = a1 + 0.5 * h
    ncx = cx + dx * w
    ncy = cy + dy * h
    half_nw = 0.5 * (w * jnp.exp(dw))
    half_nh = 0.5 * (h * jnp.exp(dh))
    out = jnp.concatenate(
        [ncx - half_nw, ncy - half_nh, ncx + half_nw, ncy + half_nh], axis=-1
    )
    out_ref[0] = jnp.clip(out, 0.0, 1.0)


def kernel(logits, regress, anchors):
    n = regress.shape[1]
    grid = (n + _BLOCK_N - 1) // _BLOCK_N
    boxes = pl.pallas_call(
        _decode_body,
        grid=(grid,),
        in_specs=[
            pl.BlockSpec((1, _BLOCK_N, 4), lambda i: (0, i, 0)),
            pl.BlockSpec((_BLOCK_N, 4), lambda i: (i, 0)),
        ],
        out_specs=pl.BlockSpec((1, _BLOCK_N, 4), lambda i: (0, i, 0)),
        out_shape=jax.ShapeDtypeStruct(regress.shape, regress.dtype),
    )(regress, anchors)
    return (logits, boxes)
